# Initial kernel scaffold; baseline (speedup 1.0000x reference)
import jax
import jax.numpy as jnp
from jax.experimental import pallas as pl


def kernel(pred, connectivity, elem_directions, elem_lengths, prop_E, prop_A, prop_I22, line_load, bc_disp, bc_rot):
    raise NotImplementedError("write your pallas kernel here")


# speedup vs baseline: 4.5284x; 4.5284x over previous
import functools

import jax
import jax.numpy as jnp
from jax import lax
from jax.experimental import pallas as pl
from jax.experimental.pallas import tpu as pltpu
from jax.experimental.pallas import tpu_sc as plsc

NC = 2
NS = 16
NW = NC * NS
C = 800
SUB = 100
NSUB = C // SUB
ZR = 250


def _rsqrt16(x):
    i = plsc.bitcast(x, jnp.int32)
    i = jnp.int32(0x5F3759DF) - lax.shift_right_logical(i, 1)
    y = plsc.bitcast(i, jnp.float32)
    for _ in range(3):
        y = y * (1.5 - 0.5 * x * y * y)
    return y


def _normalize3(v0, v1, v2):
    vv = v0 * v0 + v1 * v1 + v2 * v2
    vvc = jnp.maximum(vv, 1e-30)
    r = _rsqrt16(vvc)
    nrm = vvc * r
    inv = 1.0 / jnp.maximum(nrm, 1e-8)
    return v0 * inv, v1 * inv, v2 * inv


def _make_edge_pass(N, E):
    PW = E // NW
    nfull, rem = divmod(PW, C)
    assert E % NW == 0 and PW % 8 == 0
    nzr = (N // NS) // ZR
    assert N % (NS * ZR) == 0

    iota = lambda: lax.iota(jnp.int32, 16)

    def body(pred_h, conn_h, dirs_h, l_h, pe_h, pa_h, pi_h,
             accout_h, sums_h,
             conn_v, dirs_v, l_v, pe_v, pa_v, pi_v,
             idx_i, idx_j, rows_i, rows_j, ci_v, cj_v, zb_v, sb_v,
             acc_s, sem):
        cid = lax.axis_index("c")
        sid = lax.axis_index("s")
        wid = cid * NS + sid

        zf = jnp.zeros((16,), jnp.float32)

        def zb_body(k, _):
            fl = k * 16 + iota()
            r8 = lax.shift_right_logical(fl, 3)
            c8 = lax.bitwise_and(fl, 7)
            plsc.store_scatter(zb_v, [r8, c8], zf)
            return 0
        lax.fori_loop(0, (ZR * 8) // 16, zb_body, 0)

        def zc_body(k, _):
            fl = k * 16 + iota()
            r8 = lax.shift_right_logical(fl, 3)
            c8 = lax.bitwise_and(fl, 7)
            plsc.store_scatter(ci_v, [r8, c8], zf)
            plsc.store_scatter(cj_v, [r8, c8], zf)
            return 0
        lax.fori_loop(0, (C * 8) // 16, zc_body, 0)

        def zacc_body(t, _):
            pltpu.sync_copy(zb_v, acc_s.at[pl.ds(sid * (N // NS) + t * ZR, ZR)])
            return 0
        lax.fori_loop(0, nzr, zacc_body, 0)
        plsc.subcore_barrier()

        def do_chunk(eb, mask_start, sea, seil):
            pltpu.sync_copy(conn_h.at[pl.ds(eb, C)], conn_v)
            pltpu.sync_copy(dirs_h.at[pl.ds(eb, C)], dirs_v)
            pltpu.sync_copy(l_h.at[pl.ds(eb, C)], l_v)
            pltpu.sync_copy(pe_h.at[pl.ds(eb, C)], pe_v)
            pltpu.sync_copy(pa_h.at[pl.ds(eb, C)], pa_v)
            pltpu.sync_copy(pi_h.at[pl.ds(eb, C)], pi_v)

            c0 = jnp.full((16,), 0, jnp.int32)
            c1 = jnp.full((16,), 1, jnp.int32)

            def idx_body(k, _):
                fl = k * 16 + iota()
                vi = plsc.load_gather(conn_v, [fl, c0])
                vj = plsc.load_gather(conn_v, [fl, c1])
                r = lax.div(fl, SUB)
                cc = fl - r * SUB
                plsc.store_scatter(idx_i, [r, cc], vi)
                plsc.store_scatter(idx_j, [r, cc], vj)
                return 0
            lax.fori_loop(0, C // 16, idx_body, 0)

            descs = []
            for j in range(NSUB):
                descs.append(pltpu.async_copy(
                    pred_h.at[idx_i.at[j]], rows_i.at[pl.ds(j * SUB, SUB)], sem))
                descs.append(pltpu.async_copy(
                    pred_h.at[idx_j.at[j]], rows_j.at[pl.ds(j * SUB, SUB)], sem))
            for d in descs:
                d.wait()

            def compute_body(k, carry):
                sea_c, seil_c = carry
                fl = k * 16 + iota()
                colv = [jnp.full((16,), c, jnp.int32) for c in range(6)]
                x0 = plsc.load_gather(dirs_v, [fl, colv[0]])
                x1 = plsc.load_gather(dirs_v, [fl, colv[1]])
                x2 = plsc.load_gather(dirs_v, [fl, colv[2]])
                off = k * 16
                lv = l_v[pl.ds(off, 16)]
                pev = pe_v[pl.ds(off, 16)]
                pav = pa_v[pl.ds(off, 16)]
                piv = pi_v[pl.ds(off, 16)]
                ui0 = plsc.load_gather(rows_i, [fl, colv[0]])
                ui1 = plsc.load_gather(rows_i, [fl, colv[1]])
                ui2 = plsc.load_gather(rows_i, [fl, colv[2]])
                ti0 = plsc.load_gather(rows_i, [fl, colv[3]])
                ti1 = plsc.load_gather(rows_i, [fl, colv[4]])
                ti2 = plsc.load_gather(rows_i, [fl, colv[5]])
                uj0 = plsc.load_gather(rows_j, [fl, colv[0]])
                uj1 = plsc.load_gather(rows_j, [fl, colv[1]])
                uj2 = plsc.load_gather(rows_j, [fl, colv[2]])
                tj0 = plsc.load_gather(rows_j, [fl, colv[3]])
                tj1 = plsc.load_gather(rows_j, [fl, colv[4]])
                tj2 = plsc.load_gather(rows_j, [fl, colv[5]])

                par = jnp.abs(x1) > 0.99
                z0 = jnp.where(par, x1, -x2)
                z1 = jnp.where(par, -x0, 0.0)
                z2 = jnp.where(par, 0.0, x0)
                z0, z1, z2 = _normalize3(z0, z1, z2)
                y0 = z1 * x2 - z2 * x1
                y1 = z2 * x0 - z0 * x2
                y2 = z0 * x1 - z1 * x0
                y0, y1, y2 = _normalize3(y0, y1, y2)

                du0 = uj0 - ui0
                du1 = uj1 - ui1
                du2 = uj2 - ui2
                inv_l = 1.0 / lv
                ea = pev * pav
                ei = pev * piv
                axial = du0 * x0 + du1 * x1 + du2 * x2
                na = ea * inv_l * axial
                inv_l2 = inv_l * inv_l
                a12 = 12.0 * ei * inv_l2 * inv_l
                a6 = 6.0 * ei * inv_l2
                al = ei * inv_l
                dwz = du0 * z0 + du1 * z1 + du2 * z2
                tyi = ti0 * y0 + ti1 * y1 + ti2 * y2
                tyj = tj0 * y0 + tj1 * y1 + tj2 * y2
                vz = a12 * dwz + a6 * (tyi + tyj)
                myi = a6 * dwz + al * (2.0 * tyi + tyj)
                myj = a6 * dwz + al * (tyi + 2.0 * tyj)
                dwy = du0 * y0 + du1 * y1 + du2 * y2
                tzi = ti0 * z0 + ti1 * z1 + ti2 * z2
                tzj = tj0 * z0 + tj1 * z1 + tj2 * z2
                vy = a12 * dwy + a6 * (tzi + tzj)
                mzi = a6 * dwy + al * (2.0 * tzi + tzj)
                mzj = a6 * dwy + al * (tzi + 2.0 * tzj)
                f0 = na * x0 + vz * z0 + vy * y0
                f1 = na * x1 + vz * z1 + vy * y1
                f2 = na * x2 + vz * z2 + vy * y2
                mi0 = myi * y0 + mzi * z0
                mi1 = myi * y1 + mzi * z1
                mi2 = myi * y2 + mzi * z2
                mj0 = myj * y0 + mzj * z0
                mj1 = myj * y1 + mzj * z1
                mj2 = myj * y2 + mzj * z2
                d_ea = ea
                d_eil = al
                if mask_start:
                    fm = jnp.where(fl >= mask_start, 1.0, 0.0)
                    f0, f1, f2 = f0 * fm, f1 * fm, f2 * fm
                    mi0, mi1, mi2 = mi0 * fm, mi1 * fm, mi2 * fm
                    mj0, mj1, mj2 = mj0 * fm, mj1 * fm, mj2 * fm
                    d_ea = d_ea * fm
                    d_eil = d_eil * fm
                plsc.store_scatter(ci_v, [fl, colv[0]], f0)
                plsc.store_scatter(ci_v, [fl, colv[1]], f1)
                plsc.store_scatter(ci_v, [fl, colv[2]], f2)
                plsc.store_scatter(ci_v, [fl, colv[3]], mi0)
                plsc.store_scatter(ci_v, [fl, colv[4]], mi1)
                plsc.store_scatter(ci_v, [fl, colv[5]], mi2)
                plsc.store_scatter(cj_v, [fl, colv[0]], -f0)
                plsc.store_scatter(cj_v, [fl, colv[1]], -f1)
                plsc.store_scatter(cj_v, [fl, colv[2]], -f2)
                plsc.store_scatter(cj_v, [fl, colv[3]], mj0)
                plsc.store_scatter(cj_v, [fl, colv[4]], mj1)
                plsc.store_scatter(cj_v, [fl, colv[5]], mj2)
                return (sea_c + d_ea, seil_c + d_eil)

            sea, seil = lax.fori_loop(0, C // 16, compute_body, (sea, seil))

            for j in range(NSUB):
                pltpu.sync_copy(ci_v.at[pl.ds(j * SUB, SUB)],
                                acc_s.at[idx_i.at[j]], add=True)
                pltpu.sync_copy(cj_v.at[pl.ds(j * SUB, SUB)],
                                acc_s.at[idx_j.at[j]], add=True)
            return sea, seil

        base = wid * PW

        def chunk_body(g, carry):
            return do_chunk(base + g * C, 0, *carry)

        sea, seil = lax.fori_loop(0, nfull, chunk_body, (zf, zf))
        if rem:
            sea, seil = do_chunk(base + PW - C, C - rem, sea, seil)

        plsc.subcore_barrier()

        sb_v[pl.ds(0, 16)] = sea
        sb_v[pl.ds(16, 16)] = seil
        pltpu.sync_copy(sb_v, sums_h.at[wid])

        def out_body(t, _):
            r0 = sid * (N // NS) + t * ZR
            pltpu.sync_copy(acc_s.at[pl.ds(r0, ZR)], zb_v)
            pltpu.sync_copy(zb_v, accout_h.at[cid, pl.ds(r0, ZR)])
            return 0
        lax.fori_loop(0, nzr, out_body, 0)

    mesh = plsc.VectorSubcoreMesh(core_axis_name="c", subcore_axis_name="s",
                                  num_cores=NC, num_subcores=NS)
    return pl.kernel(
        body,
        out_type=[jax.ShapeDtypeStruct((NC, N, 8), jnp.float32),
                  jax.ShapeDtypeStruct((NW, 32), jnp.float32)],
        mesh=mesh,
        compiler_params=pltpu.CompilerParams(use_tc_tiling_on_sc=False,
                                             needs_layout_passes=False),
        scratch_types=[
            pltpu.VMEM((C, 2), jnp.int32),
            pltpu.VMEM((C, 3), jnp.float32),
            pltpu.VMEM((C,), jnp.float32),
            pltpu.VMEM((C,), jnp.float32),
            pltpu.VMEM((C,), jnp.float32),
            pltpu.VMEM((C,), jnp.float32),
            pltpu.VMEM((NSUB, SUB), jnp.int32),
            pltpu.VMEM((NSUB, SUB), jnp.int32),
            pltpu.VMEM((C, 8), jnp.float32),
            pltpu.VMEM((C, 8), jnp.float32),
            pltpu.VMEM((C, 8), jnp.float32),
            pltpu.VMEM((C, 8), jnp.float32),
            pltpu.VMEM((ZR, 8), jnp.float32),
            pltpu.VMEM((32,), jnp.float32),
            pltpu.VMEM_SHARED((N, 8), jnp.float32),
            pltpu.SemaphoreType.DMA,
        ],
    )


def _make_loss_pass(N, E):
    inv_e = 1.0 / float(E)

    def body(acc_ref, sums_ref, ll_ref, bcd_ref, bcr_ref, out_ref):
        sums = sums_ref[...]
        ea_mean = jnp.sum(sums[:, 0:16]) * inv_e
        eil_mean = jnp.sum(sums[:, 16:32]) * inv_e
        inv_f = 1.0 / jnp.maximum(ea_mean, 1.0)
        inv_m = 1.0 / jnp.maximum(eil_mean, 1.0)
        free_d = bcd_ref[...] < 0.5
        free_r = bcr_ref[...] < 0.5
        acc = acc_ref[...]
        ll = ll_ref[...]
        fsum = jnp.zeros((), jnp.float32)
        msum = jnp.zeros((), jnp.float32)
        for c in range(3):
            fres = (acc[c:c + 1, :] + acc[8 + c:9 + c, :] + ll[c:c + 1, :]) * inv_f
            mres = (acc[3 + c:4 + c, :] + acc[11 + c:12 + c, :]) * inv_m
            fsum = fsum + jnp.sum(jnp.where(free_d, fres * fres, 0.0))
            msum = msum + jnp.sum(jnp.where(free_r, mres * mres, 0.0))
        nf = jnp.sum(jnp.where(free_d, 1.0, 0.0)) * 3.0
        nm = jnp.sum(jnp.where(free_r, 1.0, 0.0)) * 3.0
        out_ref[0, 0] = fsum / nf + msum / nm

    return pl.pallas_call(
        body,
        out_shape=jax.ShapeDtypeStruct((1, 1), jnp.float32),
        out_specs=pl.BlockSpec(memory_space=pltpu.SMEM),
    )


def kernel(pred, connectivity, elem_directions, elem_lengths, prop_E, prop_A,
           prop_I22, line_load, bc_disp, bc_rot):
    N = pred.shape[0]
    E = connectivity.shape[0]
    pred8 = jnp.concatenate(
        [pred, jnp.zeros((N, 2), pred.dtype)], axis=1)
    accout, sums = _make_edge_pass(N, E)(
        pred8, connectivity, elem_directions, elem_lengths,
        prop_E, prop_A, prop_I22)
    acc_t = accout.transpose(0, 2, 1).reshape(2 * 8, N)
    loss = _make_loss_pass(N, E)(
        acc_t, sums, line_load.T, bc_disp.T, bc_rot.T)
    return loss[0, 0]
